# baseline (speedup 1.0000x reference)
import functools

import jax
import jax.numpy as jnp
from jax import lax
from jax.experimental import pallas as pl
from jax.experimental.pallas import tpu as pltpu
from jax.experimental.pallas import tpu_sc as plsc

_DIMS = (((1,), (0,)), ((), ()))


def _dot_bf16(a, b):
    return lax.dot_general(a.astype(jnp.bfloat16), b.astype(jnp.bfloat16),
                           _DIMS, preferred_element_type=jnp.float32)


def _knn_body(n2, tiles_per_batch, x1_ref, x2t_ref, aux_ref, gidx_ref, w_ref):
    step = pl.program_id(0)

    x1 = x1_ref[...]
    x2t = x2t_ref[0, 0:3, :]
    x1sq = jnp.sum(x1 * x1, axis=1, keepdims=True)
    x2sq = jnp.sum(x2t * x2t, axis=0, keepdims=True)
    d2 = (x1sq + x2sq) + _dot_bf16(x1 * jnp.float32(-2.0), x2t)

    T = x1.shape[0]
    iota = lax.broadcasted_iota(jnp.int32, (T, n2), 1).astype(jnp.float32)
    n2f = jnp.float32(n2)
    big = jnp.float32(3.4e38)
    aux = aux_ref[...]

    def mxu_argmin(d, m):
        oh = jnp.where(d == m, 1.0, 0.0).astype(jnp.bfloat16)
        s = lax.dot_general(oh, aux, _DIMS,
                            preferred_element_type=jnp.float32)
        am = s[:, 0:1] * 8.0 + s[:, 1:2]
        return am, s[:, 2:3]

    m0 = jnp.min(d2, axis=1, keepdims=True)
    am0, c0 = mxu_argmin(d2, m0)
    dm = jnp.where(iota == am0, big, d2)
    m1 = jnp.min(dm, axis=1, keepdims=True)
    am1, c1 = mxu_argmin(dm, m1)
    dm2 = jnp.where(iota == am1, big, dm)
    m2 = jnp.min(dm2, axis=1, keepdims=True)
    am2, c2 = mxu_argmin(dm2, m2)

    cmax = jnp.maximum(jnp.max(c0), jnp.maximum(jnp.max(c1), jnp.max(c2)))

    bofs = (step // tiles_per_batch) * n2

    def emit(vals, idxs_f):
        recips = [1.0 / (jnp.sqrt(jnp.maximum(v, 0.0)) + 1e-8) for v in vals]
        norm = recips[0] + recips[1] + recips[2]
        ws = [r / norm for r in recips]
        gs = [i.astype(jnp.int32) + bofs for i in idxs_f]
        gcat = jnp.concatenate(gs + gs + gs[:2], axis=1)
        gidx_ref[0] = jnp.transpose(gcat, (1, 0))
        w_ref[...] = jnp.concatenate(ws + ws + ws[:2], axis=1)

    @pl.when(cmax < 1.5)
    def _fast():
        emit([m0, m1, m2], [am0, am1, am2])

    @pl.when(cmax >= 1.5)
    def _ties():
        b0 = jnp.min(jnp.where(d2 == m0, iota, n2f), axis=1, keepdims=True)
        e1 = jnp.where(iota == b0, big, d2)
        n1 = jnp.min(e1, axis=1, keepdims=True)
        b1 = jnp.min(jnp.where(e1 == n1, iota, n2f), axis=1, keepdims=True)
        e2 = jnp.where(iota == b1, big, e1)
        n2v = jnp.min(e2, axis=1, keepdims=True)
        b2 = jnp.min(jnp.where(e2 == n2v, iota, n2f), axis=1, keepdims=True)
        emit([m0, n1, n2v], [b0, b1, b2])


def _make_sc_gather(total_q, chunk, d):
    info = plsc.get_sparse_core_info()
    nc, ns = info.num_cores, info.num_subcores
    nw = nc * ns
    mesh = plsc.VectorSubcoreMesh(core_axis_name="c", subcore_axis_name="s")
    q_per_w = total_q // nw
    c_per_w = q_per_w // chunk
    n_chunks = 3 * c_per_w

    nbuf = 4

    @functools.partial(
        pl.kernel, mesh=mesh,
        out_type=jax.ShapeDtypeStruct((3 * total_q, d), jnp.float32),
        scratch_types=(
            [pltpu.VMEM((8, q_per_w), jnp.int32)]
            + [pltpu.VMEM((chunk, d), jnp.float32) for _ in range(nbuf)]
            + [pltpu.SemaphoreType.DMA for _ in range(2 * nbuf)]
        ),
    )
    def gather_kernel(gidx_hbm, table_hbm, out_hbm, idx_v, *scr):
        bufs = scr[:nbuf]
        rsems = scr[nbuf:2 * nbuf]
        wsems = scr[2 * nbuf:3 * nbuf]
        wid = lax.axis_index("s") * nc + lax.axis_index("c")
        pltpu.sync_copy(gidx_hbm.at[wid], idx_v)

        def cp_read(j):
            k, c = j // c_per_w, j % c_per_w
            return pltpu.async_copy(
                table_hbm.at[idx_v.at[k, pl.ds(c * chunk, chunk)]],
                bufs[j % nbuf], rsems[j % nbuf])

        def cp_write(j):
            k, c = j // c_per_w, j % c_per_w
            base = k * total_q + wid * q_per_w + c * chunk
            return pltpu.async_copy(bufs[j % nbuf],
                                    out_hbm.at[pl.ds(base, chunk)],
                                    wsems[j % nbuf])

        rh = [None] * n_chunks
        wh = [None] * n_chunks
        depth = nbuf - 1
        for j in range(min(depth, n_chunks)):
            rh[j] = cp_read(j)
        for j in range(n_chunks):
            rh[j].wait()
            wh[j] = cp_write(j)
            nj = j + depth
            if nj < n_chunks:
                if nj - nbuf >= 0:
                    wh[nj - nbuf].wait()
                rh[nj] = cp_read(nj)
        for j in range(max(0, n_chunks - nbuf), n_chunks):
            wh[j].wait()

    return gather_kernel, nw


def _mlp_body(total_n, tile, g_ref, w_ref, p1_ref, w1t_ref, b1_ref,
              gamma_ref, beta_ref, w2t_ref, b2_ref, out_ref,
              y1_scr, sums_scr):
    phase = pl.program_id(0)
    i = pl.program_id(1)

    @pl.when(phase == 0)
    def _phase0():
        w = w_ref[...]
        interp = (w[:, 0:1] * g_ref[0]
                  + w[:, 1:2] * g_ref[1]
                  + w[:, 2:3] * g_ref[2])
        y1 = (_dot_bf16(p1_ref[...], w1t_ref[0:64, :])
              + _dot_bf16(interp, w1t_ref[64:192, :])
              + b1_ref[...])
        y1_scr[pl.ds(i * tile, tile), :] = y1

        @pl.when(i == 0)
        def _init():
            sums_scr[...] = jnp.zeros_like(sums_scr)

        sums_scr[0:1, :] = (sums_scr[0:1, :]
                            + jnp.sum(y1, axis=0, keepdims=True))
        sums_scr[1:2, :] = (sums_scr[1:2, :]
                            + jnp.sum(y1 * y1, axis=0, keepdims=True))

    @pl.when(phase == 1)
    def _phase1():
        inv_n = jnp.float32(1.0 / total_n)
        mean = sums_scr[0:1, :] * inv_n
        var = sums_scr[1:2, :] * inv_n - mean * mean
        scale = gamma_ref[...] / jnp.sqrt(var + 1e-5)
        y = (y1_scr[pl.ds(i * tile, tile), :] - mean) * scale + beta_ref[...]
        y = jnp.maximum(y, 0.0)
        out_ref[...] = _dot_bf16(y, w2t_ref[...]) + b2_ref[...]


def kernel(xyz1, xyz2, points1, points2, W1, b1, gamma, beta, W2, b2):
    B, N1, _ = xyz1.shape
    _, N2, _ = xyz2.shape
    C1 = points1.shape[-1]
    C2 = points2.shape[-1]
    Cout = W1.shape[0]
    T = 512
    tiles_per_batch = N1 // T
    n_tiles = B * tiles_per_batch
    total_n = B * N1
    CHUNK = 128

    x1r = xyz1.reshape(B * N1, 3)
    x2t = jnp.pad(jnp.transpose(xyz2, (0, 2, 1)), ((0, 0), (0, 5), (0, 0)))
    ii = jnp.arange(N2, dtype=jnp.int32)
    aux = jnp.stack(
        [ii // 8, ii % 8, jnp.ones((N2,), jnp.int32)]
        + [jnp.zeros((N2,), jnp.int32)] * 5, axis=1).astype(jnp.bfloat16)
    p1 = points1.reshape(B * N1, C1)
    w1t = W1.T
    w2t = W2.T
    b1r = b1.reshape(1, Cout)
    b2r = b2.reshape(1, Cout)
    gr = gamma.reshape(1, Cout)
    br = beta.reshape(1, Cout)

    n_workers = B * N1 // T
    gidx_t, w = pl.pallas_call(
        functools.partial(_knn_body, N2, tiles_per_batch),
        grid=(n_tiles,),
        in_specs=[
            pl.BlockSpec((T, 3), lambda i: (i, 0)),
            pl.BlockSpec((1, 8, N2), lambda i, tpb=tiles_per_batch: (i // tpb, 0, 0)),
            pl.BlockSpec((N2, 8), lambda i: (0, 0)),
        ],
        out_specs=[
            pl.BlockSpec((1, 8, T), lambda i: (i, 0, 0)),
            pl.BlockSpec((T, 8), lambda i: (i, 0)),
        ],
        out_shape=[
            jax.ShapeDtypeStruct((n_workers, 8, T), jnp.int32),
            jax.ShapeDtypeStruct((B * N1, 8), jnp.float32),
        ],
    )(x1r, x2t, aux)

    gather_kernel, nw = _make_sc_gather(B * N1, CHUNK, C2)
    table = points2.reshape(B * N2, C2)

    gathered = gather_kernel(gidx_t, table)
    g3 = gathered.reshape(3, B * N1, C2)

    out = pl.pallas_call(
        functools.partial(_mlp_body, total_n, T),
        grid=(2, n_tiles),
        in_specs=[
            pl.BlockSpec((3, T, C2), lambda p, i: (0, jnp.where(p == 0, i, 0), 0)),
            pl.BlockSpec((T, 8), lambda p, i: (jnp.where(p == 0, i, 0), 0)),
            pl.BlockSpec((T, C1), lambda p, i: (jnp.where(p == 0, i, 0), 0)),
            pl.BlockSpec((C1 + C2, Cout), lambda p, i: (0, 0)),
            pl.BlockSpec((1, Cout), lambda p, i: (0, 0)),
            pl.BlockSpec((1, Cout), lambda p, i: (0, 0)),
            pl.BlockSpec((1, Cout), lambda p, i: (0, 0)),
            pl.BlockSpec((Cout, Cout), lambda p, i: (0, 0)),
            pl.BlockSpec((1, Cout), lambda p, i: (0, 0)),
        ],
        out_specs=pl.BlockSpec((T, Cout), lambda p, i: (jnp.where(p == 0, 0, i), 0)),
        out_shape=jax.ShapeDtypeStruct((B * N1, Cout), jnp.float32),
        scratch_shapes=[
            pltpu.VMEM((B * N1, Cout), jnp.float32),
            pltpu.VMEM((8, Cout), jnp.float32),
        ],
    )(g3, w, p1, w1t, b1r, gr, br, w2t, b2r)

    return out.reshape(B, N1, Cout)


# speedup vs baseline: 1.5446x; 1.5446x over previous
import functools

import jax
import jax.numpy as jnp
from jax import lax
from jax.experimental import pallas as pl
from jax.experimental.pallas import tpu as pltpu
from jax.experimental.pallas import tpu_sc as plsc

_DIMS = (((1,), (0,)), ((), ()))


def _dot_bf16(a, b):
    return lax.dot_general(a.astype(jnp.bfloat16), b.astype(jnp.bfloat16),
                           _DIMS, preferred_element_type=jnp.float32)


def _knn_body(n2, tiles_per_batch, x1_ref, x2t_ref, gidx_ref, w_ref):
    step = pl.program_id(0)

    x1 = x1_ref[...]
    x2t = x2t_ref[0, 0:3, :]
    x1sq = jnp.sum(x1 * x1, axis=1, keepdims=True)
    x2sq = jnp.sum(x2t * x2t, axis=0, keepdims=True)
    d2 = (x1sq + x2sq) + _dot_bf16(x1 * jnp.float32(-2.0), x2t)

    T = x1.shape[0]
    iota = lax.broadcasted_iota(jnp.int32, (T, n2), 1).astype(jnp.float32)
    n2f = jnp.float32(n2)
    big = jnp.float32(3.4e38)

    m0 = jnp.min(d2, axis=1, keepdims=True)
    am0 = jnp.min(jnp.where(d2 == m0, iota, n2f), axis=1, keepdims=True)
    dm = jnp.where(iota == am0, big, d2)
    m1 = jnp.min(dm, axis=1, keepdims=True)
    am1 = jnp.min(jnp.where(dm == m1, iota, n2f), axis=1, keepdims=True)
    dm2 = jnp.where(iota == am1, big, dm)
    m2 = jnp.min(dm2, axis=1, keepdims=True)
    am2 = jnp.min(jnp.where(dm2 == m2, iota, n2f), axis=1, keepdims=True)
    vals = [m0, m1, m2]
    idxs = [am0.astype(jnp.int32), am1.astype(jnp.int32),
            am2.astype(jnp.int32)]

    recips = [1.0 / (jnp.sqrt(jnp.maximum(v, 0.0)) + 1e-8) for v in vals]
    norm = recips[0] + recips[1] + recips[2]
    ws = [r / norm for r in recips]

    bofs = (step // tiles_per_batch) * n2
    gs = [i + bofs for i in idxs]
    gcat = jnp.concatenate(gs + gs + gs[:2], axis=1)
    gidx_ref[0] = jnp.transpose(gcat, (1, 0))
    w_ref[...] = jnp.concatenate(ws + ws + ws[:2], axis=1)


def _make_sc_gather(total_q, chunk, d):
    info = plsc.get_sparse_core_info()
    nc, ns = info.num_cores, info.num_subcores
    nw = nc * ns
    mesh = plsc.VectorSubcoreMesh(core_axis_name="c", subcore_axis_name="s")
    q_per_w = total_q // nw
    c_per_w = q_per_w // chunk
    n_chunks = 3 * c_per_w

    nbuf = 4

    @functools.partial(
        pl.kernel, mesh=mesh,
        out_type=jax.ShapeDtypeStruct((3 * total_q, d), jnp.float32),
        scratch_types=(
            [pltpu.VMEM((8, q_per_w), jnp.int32)]
            + [pltpu.VMEM((chunk, d), jnp.float32) for _ in range(nbuf)]
            + [pltpu.SemaphoreType.DMA for _ in range(2 * nbuf)]
        ),
    )
    def gather_kernel(gidx_hbm, table_hbm, out_hbm, idx_v, *scr):
        bufs = scr[:nbuf]
        rsems = scr[nbuf:2 * nbuf]
        wsems = scr[2 * nbuf:3 * nbuf]
        wid = lax.axis_index("s") * nc + lax.axis_index("c")
        pltpu.sync_copy(gidx_hbm.at[wid], idx_v)

        def cp_read(j):
            k, c = j // c_per_w, j % c_per_w
            return pltpu.async_copy(
                table_hbm.at[idx_v.at[k, pl.ds(c * chunk, chunk)]],
                bufs[j % nbuf], rsems[j % nbuf])

        def cp_write(j):
            k, c = j // c_per_w, j % c_per_w
            base = k * total_q + wid * q_per_w + c * chunk
            return pltpu.async_copy(bufs[j % nbuf],
                                    out_hbm.at[pl.ds(base, chunk)],
                                    wsems[j % nbuf])

        rh = [None] * n_chunks
        wh = [None] * n_chunks
        depth = nbuf - 1
        for j in range(min(depth, n_chunks)):
            rh[j] = cp_read(j)
        for j in range(n_chunks):
            rh[j].wait()
            wh[j] = cp_write(j)
            nj = j + depth
            if nj < n_chunks:
                if nj - nbuf >= 0:
                    wh[nj - nbuf].wait()
                rh[nj] = cp_read(nj)
        for j in range(max(0, n_chunks - nbuf), n_chunks):
            wh[j].wait()

    return gather_kernel, nw


def _mlp_body(total_n, tile, g_ref, w_ref, p1_ref, w1t_ref, b1_ref,
              gamma_ref, beta_ref, w2t_ref, b2_ref, out_ref,
              y1_scr, sums_scr):
    phase = pl.program_id(0)
    i = pl.program_id(1)

    @pl.when(phase == 0)
    def _phase0():
        w = w_ref[...]
        interp = (w[:, 0:1] * g_ref[0]
                  + w[:, 1:2] * g_ref[1]
                  + w[:, 2:3] * g_ref[2])
        y1 = (_dot_bf16(p1_ref[...], w1t_ref[0:64, :])
              + _dot_bf16(interp, w1t_ref[64:192, :])
              + b1_ref[...])
        y1_scr[pl.ds(i * tile, tile), :] = y1

        @pl.when(i == 0)
        def _init():
            sums_scr[...] = jnp.zeros_like(sums_scr)

        sums_scr[0:1, :] = (sums_scr[0:1, :]
                            + jnp.sum(y1, axis=0, keepdims=True))
        sums_scr[1:2, :] = (sums_scr[1:2, :]
                            + jnp.sum(y1 * y1, axis=0, keepdims=True))

    @pl.when(phase == 1)
    def _phase1():
        inv_n = jnp.float32(1.0 / total_n)
        mean = sums_scr[0:1, :] * inv_n
        var = sums_scr[1:2, :] * inv_n - mean * mean
        scale = gamma_ref[...] / jnp.sqrt(var + 1e-5)
        y = (y1_scr[pl.ds(i * tile, tile), :] - mean) * scale + beta_ref[...]
        y = jnp.maximum(y, 0.0)
        out_ref[...] = _dot_bf16(y, w2t_ref[...]) + b2_ref[...]


def kernel(xyz1, xyz2, points1, points2, W1, b1, gamma, beta, W2, b2):
    B, N1, _ = xyz1.shape
    _, N2, _ = xyz2.shape
    C1 = points1.shape[-1]
    C2 = points2.shape[-1]
    Cout = W1.shape[0]
    T = 512
    tiles_per_batch = N1 // T
    n_tiles = B * tiles_per_batch
    total_n = B * N1
    CHUNK = 128

    x1r = xyz1.reshape(B * N1, 3)
    x2t = jnp.pad(jnp.transpose(xyz2, (0, 2, 1)), ((0, 0), (0, 5), (0, 0)))
    p1 = points1.reshape(B * N1, C1)
    w1t = W1.T
    w2t = W2.T
    b1r = b1.reshape(1, Cout)
    b2r = b2.reshape(1, Cout)
    gr = gamma.reshape(1, Cout)
    br = beta.reshape(1, Cout)

    n_workers = B * N1 // T
    gidx_t, w = pl.pallas_call(
        functools.partial(_knn_body, N2, tiles_per_batch),
        grid=(n_tiles,),
        in_specs=[
            pl.BlockSpec((T, 3), lambda i: (i, 0)),
            pl.BlockSpec((1, 8, N2), lambda i, tpb=tiles_per_batch: (i // tpb, 0, 0)),
        ],
        out_specs=[
            pl.BlockSpec((1, 8, T), lambda i: (i, 0, 0)),
            pl.BlockSpec((T, 8), lambda i: (i, 0)),
        ],
        out_shape=[
            jax.ShapeDtypeStruct((n_workers, 8, T), jnp.int32),
            jax.ShapeDtypeStruct((B * N1, 8), jnp.float32),
        ],
    )(x1r, x2t)

    gather_kernel, nw = _make_sc_gather(B * N1, CHUNK, C2)
    table = points2.reshape(B * N2, C2)

    gathered = gather_kernel(gidx_t, table)
    g3 = gathered.reshape(3, B * N1, C2)

    out = pl.pallas_call(
        functools.partial(_mlp_body, total_n, T),
        grid=(2, n_tiles),
        in_specs=[
            pl.BlockSpec((3, T, C2), lambda p, i: (0, jnp.where(p == 0, i, 0), 0)),
            pl.BlockSpec((T, 8), lambda p, i: (jnp.where(p == 0, i, 0), 0)),
            pl.BlockSpec((T, C1), lambda p, i: (jnp.where(p == 0, i, 0), 0)),
            pl.BlockSpec((C1 + C2, Cout), lambda p, i: (0, 0)),
            pl.BlockSpec((1, Cout), lambda p, i: (0, 0)),
            pl.BlockSpec((1, Cout), lambda p, i: (0, 0)),
            pl.BlockSpec((1, Cout), lambda p, i: (0, 0)),
            pl.BlockSpec((Cout, Cout), lambda p, i: (0, 0)),
            pl.BlockSpec((1, Cout), lambda p, i: (0, 0)),
        ],
        out_specs=pl.BlockSpec((T, Cout), lambda p, i: (jnp.where(p == 0, 0, i), 0)),
        out_shape=jax.ShapeDtypeStruct((B * N1, Cout), jnp.float32),
        scratch_shapes=[
            pltpu.VMEM((B * N1, Cout), jnp.float32),
            pltpu.VMEM((8, Cout), jnp.float32),
        ],
    )(g3, w, p1, w1t, b1r, gr, br, w2t, b2r)

    return out.reshape(B, N1, Cout)
